# Initial kernel scaffold; baseline (speedup 1.0000x reference)
import jax
import jax.numpy as jnp
from jax.experimental import pallas as pl


def kernel(x, edge_index, W1, b1, W2, b2, W3, b3):
    raise NotImplementedError("write your pallas kernel here")


# speedup vs baseline: 4.8223x; 4.8223x over previous
import functools

import jax
import jax.numpy as jnp
from jax import lax
from jax.experimental import pallas as pl
from jax.experimental.pallas import tpu as pltpu
from jax.experimental.pallas import tpu_sc as plsc

_NC = 2
_NS = 16
_NW = _NC * _NS
_CHUNK = 128
_N_ACC = 10240
_H = 128


def _sc_aggregate(e_pad, with_gather):
    epw = e_pad // _NW
    nchunk = epw // _CHUNK
    rpt = _N_ACC // _NS
    mesh = plsc.VectorSubcoreMesh(core_axis_name="c", subcore_axis_name="s")

    scratch = [
        pltpu.VMEM((_CHUNK,), jnp.int32),
        pltpu.VMEM((_CHUNK,), jnp.int32),
        pltpu.VMEM((_CHUNK, _H), jnp.float32),
        pltpu.VMEM_SHARED((_N_ACC, _H), jnp.float32),
        pltpu.SemaphoreType.DMA,
    ]

    def _body(tab_hbm, src_hbm, dst_hbm, zero_hbm, out_hbm,
              src_v, dst_v, rows_v, acc_sh, sem):
        cid = lax.axis_index("c")
        sid = lax.axis_index("s")
        wid = sid * _NC + cid
        r0 = sid * rpt
        pltpu.sync_copy(zero_hbm.at[pl.ds(r0, rpt)], acc_sh.at[pl.ds(r0, rpt)])
        if not with_gather:
            def orow(r, carry):
                for c in range(_H // 16):
                    rows_v[r, pl.ds(c * 16, 16)] = jnp.ones((16,), jnp.float32)
                return carry

            lax.fori_loop(0, _CHUNK, orow, 0)
        plsc.subcore_barrier()
        wbase = wid * epw

        def body(i, carry):
            base = wbase + i * _CHUNK
            pltpu.sync_copy(dst_hbm.at[pl.ds(base, _CHUNK)], dst_v)
            if with_gather:
                pltpu.sync_copy(src_hbm.at[pl.ds(base, _CHUNK)], src_v)
                pltpu.async_copy(tab_hbm.at[src_v], rows_v, sem).wait()
            pltpu.sync_copy(rows_v, acc_sh.at[dst_v], add=True)
            return carry

        lax.fori_loop(0, nchunk, body, 0)
        plsc.subcore_barrier()
        pltpu.sync_copy(acc_sh.at[pl.ds(r0, rpt)],
                        out_hbm.at[pl.ds(cid * _N_ACC + r0, rpt)])

    out_type = jax.ShapeDtypeStruct((_NC * _N_ACC, _H), jnp.float32)
    if with_gather:
        @functools.partial(pl.kernel, mesh=mesh, out_type=out_type,
                           scratch_types=scratch)
        def agg(tab_hbm, src_hbm, dst_hbm, zero_hbm, out_hbm,
                src_v, dst_v, rows_v, acc_sh, sem):
            _body(tab_hbm, src_hbm, dst_hbm, zero_hbm, out_hbm,
                  src_v, dst_v, rows_v, acc_sh, sem)
    else:
        @functools.partial(pl.kernel, mesh=mesh, out_type=out_type,
                           scratch_types=scratch)
        def agg(dst_hbm, zero_hbm, out_hbm,
                src_v, dst_v, rows_v, acc_sh, sem):
            _body(None, None, dst_hbm, zero_hbm, out_hbm,
                  src_v, dst_v, rows_v, acc_sh, sem)

    return agg


def _dinv(c0_ref, c1_ref):
    cnt = c0_ref[...] + c1_ref[...] + 1.0
    return lax.rsqrt(cnt)


def _tc_first(x, w, c0, c1, blk):
    n, d = x.shape
    hn = w.shape[1]

    def body(x_ref, w_ref, c0_ref, c1_ref, o_ref):
        dinv = _dinv(c0_ref, c1_ref)
        o_ref[...] = jnp.dot(x_ref[...] * dinv, w_ref[...],
                             preferred_element_type=jnp.float32)

    return pl.pallas_call(
        body,
        grid=(n // blk,),
        in_specs=[
            pl.BlockSpec((blk, d), lambda i: (i, 0)),
            pl.BlockSpec((d, hn), lambda i: (0, 0)),
            pl.BlockSpec((blk, 1), lambda i: (i, 0)),
            pl.BlockSpec((blk, 1), lambda i: (i, 0)),
        ],
        out_specs=pl.BlockSpec((blk, hn), lambda i: (i, 0)),
        out_shape=jax.ShapeDtypeStruct((n, hn), jnp.float32),
    )(x, w, c0, c1)


def _tc_mid(ra, rb, xs, c0, c1, b, w, blk):
    n, d = xs.shape
    hn = w.shape[1]

    def body(ra_ref, rb_ref, xs_ref, c0_ref, c1_ref, b_ref, w_ref, o_ref):
        dinv = _dinv(c0_ref, c1_ref)
        t = dinv * (ra_ref[...] + rb_ref[...] + xs_ref[...]) + b_ref[...]
        h = jnp.where(t > 0, t, jnp.exp(jnp.minimum(t, 0.0)) - 1.0)
        o_ref[...] = jnp.dot(h * dinv, w_ref[...],
                             preferred_element_type=jnp.float32)

    return pl.pallas_call(
        body,
        grid=(n // blk,),
        in_specs=[
            pl.BlockSpec((blk, d), lambda i: (i, 0)),
            pl.BlockSpec((blk, d), lambda i: (i, 0)),
            pl.BlockSpec((blk, d), lambda i: (i, 0)),
            pl.BlockSpec((blk, 1), lambda i: (i, 0)),
            pl.BlockSpec((blk, 1), lambda i: (i, 0)),
            pl.BlockSpec((1, d), lambda i: (0, 0)),
            pl.BlockSpec((d, hn), lambda i: (0, 0)),
        ],
        out_specs=pl.BlockSpec((blk, hn), lambda i: (i, 0)),
        out_shape=jax.ShapeDtypeStruct((n, hn), jnp.float32),
    )(ra, rb, xs, c0, c1, b, w)


def _tc_final(ra0, rb0, ra1, rb1, x0, x1, c0, c1, b, blk):
    n = x0.shape[0]

    def body(ra0_ref, rb0_ref, ra1_ref, rb1_ref, x0_ref, x1_ref,
             c0_ref, c1_ref, b_ref, o_ref):
        dinv = _dinv(c0_ref, c1_ref)
        t0 = dinv * (ra0_ref[...] + rb0_ref[...] + x0_ref[...]) + b_ref[0:1, 0:1]
        t1 = dinv * (ra1_ref[...] + rb1_ref[...] + x1_ref[...]) + b_ref[0:1, 1:2]
        m = jnp.maximum(t0, t1)
        lse = m + jnp.log(jnp.exp(t0 - m) + jnp.exp(t1 - m))
        o_ref[...] = jnp.concatenate([t0 - lse, t1 - lse], axis=1)

    col = pl.BlockSpec((blk, 1), lambda i: (i, 0))
    return pl.pallas_call(
        body,
        grid=(n // blk,),
        in_specs=[col, col, col, col, col, col, col, col,
                  pl.BlockSpec((1, 2), lambda i: (0, 0))],
        out_specs=pl.BlockSpec((blk, 2), lambda i: (i, 0)),
        out_shape=jax.ShapeDtypeStruct((n, 2), jnp.float32),
    )(ra0, rb0, ra1, rb1, x0, x1, c0, c1, b)


def kernel(x, edge_index, W1, b1, W2, b2, W3, b3):
    x = x.astype(jnp.float32)
    n = x.shape[0]
    e = edge_index.shape[1]
    grain = _NW * _CHUNK
    e_pad = ((e + grain - 1) // grain) * grain
    pad = e_pad - e
    blk = 1000

    src_p = jnp.concatenate(
        [edge_index[0].astype(jnp.int32), jnp.zeros((pad,), jnp.int32)])
    dst_p = jnp.concatenate(
        [edge_index[1].astype(jnp.int32), jnp.full((pad,), n, jnp.int32)])

    z128 = jnp.zeros((_N_ACC, _H), jnp.float32)

    agg = _sc_aggregate(e_pad, with_gather=True)

    counts = _sc_aggregate(e_pad, with_gather=False)(dst_p, z128)
    c0 = counts[:n, 0:1]
    c1 = counts[_N_ACC:_N_ACC + n, 0:1]

    xs1 = _tc_first(x, W1, c0, c1, blk)
    raw1 = agg(xs1, src_p, dst_p, z128)
    xs2 = _tc_mid(raw1[:n], raw1[_N_ACC:_N_ACC + n], xs1, c0, c1,
                  b1.reshape(1, -1), W2, blk)
    raw2 = agg(xs2, src_p, dst_p, z128)
    w3p = jnp.pad(W3, ((0, 0), (0, _H - W3.shape[1])))
    xs3 = _tc_mid(raw2[:n], raw2[_N_ACC:_N_ACC + n], xs2, c0, c1,
                  b2.reshape(1, -1), w3p, blk)
    raw3 = agg(xs3, src_p, dst_p, z128)
    return _tc_final(raw3[:n, 0:1], raw3[_N_ACC:_N_ACC + n, 0:1],
                     raw3[:n, 1:2], raw3[_N_ACC:_N_ACC + n, 1:2],
                     xs3[:, 0:1], xs3[:, 1:2], c0, c1,
                     b3.reshape(1, -1), blk)


# speedup vs baseline: 7.7009x; 1.5969x over previous
import functools

import jax
import jax.numpy as jnp
from jax import lax
from jax.experimental import pallas as pl
from jax.experimental.pallas import tpu as pltpu
from jax.experimental.pallas import tpu_sc as plsc

_NC = 2
_NS = 16
_NW = _NC * _NS
_CHUNK = 128
_N_ACC = 10240
_H = 128


def _sc_aggregate(e_pad, with_gather):
    epw = e_pad // _NW
    nchunk = epw // _CHUNK
    rpt = _N_ACC // _NS
    mesh = plsc.VectorSubcoreMesh(core_axis_name="c", subcore_axis_name="s")

    scratch = [
        pltpu.VMEM((nchunk, _CHUNK), jnp.int32),
        pltpu.VMEM((nchunk, _CHUNK), jnp.int32),
        pltpu.VMEM((_CHUNK, _H), jnp.float32),
        pltpu.VMEM((_CHUNK, _H), jnp.float32),
        pltpu.VMEM_SHARED((_N_ACC, _H), jnp.float32),
        pltpu.SemaphoreType.DMA,
        pltpu.SemaphoreType.DMA,
    ]

    def _body(tab_hbm, src_hbm, dst_hbm, zero_hbm, out_hbm,
              src_v, dst_v, buf0, buf1, acc_sh, sems):
        cid = lax.axis_index("c")
        sid = lax.axis_index("s")
        wid = sid * _NC + cid
        r0 = sid * rpt
        bufs = (buf0, buf1)
        wrow = wid * nchunk
        pltpu.sync_copy(zero_hbm.at[pl.ds(r0, rpt)], acc_sh.at[pl.ds(r0, rpt)])
        pltpu.sync_copy(dst_hbm.at[pl.ds(wrow, nchunk)], dst_v)
        if with_gather:
            pltpu.sync_copy(src_hbm.at[pl.ds(wrow, nchunk)], src_v)
            pltpu.async_copy(tab_hbm.at[src_v.at[0]], buf0, sems[0])
            pltpu.async_copy(tab_hbm.at[src_v.at[1]], buf1, sems[1])
        else:
            def orow(r, carry):
                for c in range(_H // 16):
                    buf0[r, pl.ds(c * 16, 16)] = jnp.ones((16,), jnp.float32)
                return carry

            lax.fori_loop(0, _CHUNK, orow, 0)
        plsc.subcore_barrier()

        if with_gather:
            def group(g, carry):
                for b in range(2):
                    i = g * 2 + b
                    pltpu.make_async_copy(tab_hbm.at[src_v.at[i]], bufs[b],
                                          sems[b]).wait()
                    pltpu.sync_copy(bufs[b], acc_sh.at[dst_v.at[i]], add=True)
                    pltpu.async_copy(tab_hbm.at[src_v.at[i + 2]], bufs[b],
                                     sems[b])
                return carry

            lax.fori_loop(0, nchunk // 2 - 1, group, 0)
            for b in range(2):
                i = nchunk - 2 + b
                pltpu.make_async_copy(tab_hbm.at[src_v.at[i]], bufs[b],
                                      sems[b]).wait()
                pltpu.sync_copy(bufs[b], acc_sh.at[dst_v.at[i]], add=True)
        else:
            def body(i, carry):
                pltpu.sync_copy(buf0, acc_sh.at[dst_v.at[i]], add=True)
                return carry

            lax.fori_loop(0, nchunk, body, 0)
        plsc.subcore_barrier()
        pltpu.sync_copy(acc_sh.at[pl.ds(r0, rpt)],
                        out_hbm.at[pl.ds(cid * _N_ACC + r0, rpt)])

    out_type = jax.ShapeDtypeStruct((_NC * _N_ACC, _H), jnp.float32)
    if with_gather:
        @functools.partial(pl.kernel, mesh=mesh, out_type=out_type,
                           scratch_types=scratch)
        def agg(tab_hbm, src_hbm, dst_hbm, zero_hbm, out_hbm,
                src_v, dst_v, buf0, buf1, acc_sh, sem0, sem1):
            _body(tab_hbm, src_hbm, dst_hbm, zero_hbm, out_hbm,
                  src_v, dst_v, buf0, buf1, acc_sh, (sem0, sem1))
    else:
        @functools.partial(pl.kernel, mesh=mesh, out_type=out_type,
                           scratch_types=scratch)
        def agg(dst_hbm, zero_hbm, out_hbm,
                src_v, dst_v, buf0, buf1, acc_sh, sem0, sem1):
            _body(None, None, dst_hbm, zero_hbm, out_hbm,
                  src_v, dst_v, buf0, buf1, acc_sh, (sem0, sem1))

    return agg


def _dinv(c0_ref, c1_ref):
    cnt = c0_ref[...] + c1_ref[...] + 1.0
    return lax.rsqrt(cnt)


def _tc_first(x, w, c0, c1, blk):
    n, d = x.shape
    hn = w.shape[1]

    def body(x_ref, w_ref, c0_ref, c1_ref, o_ref):
        dinv = _dinv(c0_ref, c1_ref)
        o_ref[...] = jnp.dot(x_ref[...] * dinv, w_ref[...],
                             preferred_element_type=jnp.float32)

    return pl.pallas_call(
        body,
        grid=(n // blk,),
        in_specs=[
            pl.BlockSpec((blk, d), lambda i: (i, 0)),
            pl.BlockSpec((d, hn), lambda i: (0, 0)),
            pl.BlockSpec((blk, 1), lambda i: (i, 0)),
            pl.BlockSpec((blk, 1), lambda i: (i, 0)),
        ],
        out_specs=pl.BlockSpec((blk, hn), lambda i: (i, 0)),
        out_shape=jax.ShapeDtypeStruct((n, hn), jnp.float32),
    )(x, w, c0, c1)


def _tc_mid(ra, rb, xs, c0, c1, b, w, blk):
    n, d = xs.shape
    hn = w.shape[1]

    def body(ra_ref, rb_ref, xs_ref, c0_ref, c1_ref, b_ref, w_ref, o_ref):
        dinv = _dinv(c0_ref, c1_ref)
        t = dinv * (ra_ref[...] + rb_ref[...] + xs_ref[...]) + b_ref[...]
        h = jnp.where(t > 0, t, jnp.exp(jnp.minimum(t, 0.0)) - 1.0)
        o_ref[...] = jnp.dot(h * dinv, w_ref[...],
                             preferred_element_type=jnp.float32)

    return pl.pallas_call(
        body,
        grid=(n // blk,),
        in_specs=[
            pl.BlockSpec((blk, d), lambda i: (i, 0)),
            pl.BlockSpec((blk, d), lambda i: (i, 0)),
            pl.BlockSpec((blk, d), lambda i: (i, 0)),
            pl.BlockSpec((blk, 1), lambda i: (i, 0)),
            pl.BlockSpec((blk, 1), lambda i: (i, 0)),
            pl.BlockSpec((1, d), lambda i: (0, 0)),
            pl.BlockSpec((d, hn), lambda i: (0, 0)),
        ],
        out_specs=pl.BlockSpec((blk, hn), lambda i: (i, 0)),
        out_shape=jax.ShapeDtypeStruct((n, hn), jnp.float32),
    )(ra, rb, xs, c0, c1, b, w)


def _tc_final(ra0, rb0, ra1, rb1, x0, x1, c0, c1, b, blk):
    n = x0.shape[0]

    def body(ra0_ref, rb0_ref, ra1_ref, rb1_ref, x0_ref, x1_ref,
             c0_ref, c1_ref, b_ref, o_ref):
        dinv = _dinv(c0_ref, c1_ref)
        t0 = dinv * (ra0_ref[...] + rb0_ref[...] + x0_ref[...]) + b_ref[0:1, 0:1]
        t1 = dinv * (ra1_ref[...] + rb1_ref[...] + x1_ref[...]) + b_ref[0:1, 1:2]
        m = jnp.maximum(t0, t1)
        lse = m + jnp.log(jnp.exp(t0 - m) + jnp.exp(t1 - m))
        o_ref[...] = jnp.concatenate([t0 - lse, t1 - lse], axis=1)

    col = pl.BlockSpec((blk, 1), lambda i: (i, 0))
    return pl.pallas_call(
        body,
        grid=(n // blk,),
        in_specs=[col, col, col, col, col, col, col, col,
                  pl.BlockSpec((1, 2), lambda i: (0, 0))],
        out_specs=pl.BlockSpec((blk, 2), lambda i: (i, 0)),
        out_shape=jax.ShapeDtypeStruct((n, 2), jnp.float32),
    )(ra0, rb0, ra1, rb1, x0, x1, c0, c1, b)


def kernel(x, edge_index, W1, b1, W2, b2, W3, b3):
    x = x.astype(jnp.float32)
    n = x.shape[0]
    e = edge_index.shape[1]
    grain = _NW * _CHUNK
    e_pad = ((e + grain - 1) // grain) * grain
    pad = e_pad - e
    blk = 1000

    src_p = jnp.concatenate(
        [edge_index[0].astype(jnp.int32),
         jnp.zeros((pad,), jnp.int32)]).reshape(-1, _CHUNK)
    dst_p = jnp.concatenate(
        [edge_index[1].astype(jnp.int32),
         jnp.full((pad,), n, jnp.int32)]).reshape(-1, _CHUNK)

    z128 = jnp.zeros((_N_ACC, _H), jnp.float32)

    agg = _sc_aggregate(e_pad, with_gather=True)

    counts = _sc_aggregate(e_pad, with_gather=False)(dst_p, z128)
    c0 = counts[:n, 0:1]
    c1 = counts[_N_ACC:_N_ACC + n, 0:1]

    xs1 = _tc_first(x, W1, c0, c1, blk)
    raw1 = agg(xs1, src_p, dst_p, z128)
    xs2 = _tc_mid(raw1[:n], raw1[_N_ACC:_N_ACC + n], xs1, c0, c1,
                  b1.reshape(1, -1), W2, blk)
    raw2 = agg(xs2, src_p, dst_p, z128)
    w3p = jnp.pad(W3, ((0, 0), (0, _H - W3.shape[1])))
    xs3 = _tc_mid(raw2[:n], raw2[_N_ACC:_N_ACC + n], xs2, c0, c1,
                  b2.reshape(1, -1), w3p, blk)
    raw3 = agg(xs3, src_p, dst_p, z128)
    return _tc_final(raw3[:n, 0:1], raw3[_N_ACC:_N_ACC + n, 0:1],
                     raw3[:n, 1:2], raw3[_N_ACC:_N_ACC + n, 1:2],
                     xs3[:, 0:1], xs3[:, 1:2], c0, c1,
                     b3.reshape(1, -1), blk)
